# baseline (speedup 1.0000x reference)
import functools

import jax
import jax.numpy as jnp
from jax import lax
from jax.experimental import pallas as pl
from jax.experimental.pallas import tpu as pltpu
from jax.experimental.pallas import tpu_sc as plsc

B, L, V, D, H = 16, 512, 32000, 256, 256
TCH = 256
NT = L // TCH
UNROLL = 8


@functools.lru_cache(maxsize=None)
def _make_sc_gather(n_idx, d):
    info = plsc.get_sparse_core_info()
    nw = info.num_cores * info.num_subcores
    per_w = n_idx // nw
    mesh = plsc.VectorSubcoreMesh(core_axis_name="c", subcore_axis_name="s")

    @functools.partial(
        pl.kernel,
        mesh=mesh,
        out_type=jax.ShapeDtypeStruct((n_idx, d), jnp.float32),
        scratch_types=[
            pltpu.VMEM((per_w,), jnp.int32),
            pltpu.VMEM((per_w, d), jnp.float32),
            pltpu.SemaphoreType.DMA,
        ],
    )
    def gather_k(table_hbm, idx_hbm, out_hbm, idx_v, rows_v, sem):
        wid = lax.axis_index("s") * info.num_cores + lax.axis_index("c")
        base = wid * per_w
        pltpu.sync_copy(idx_hbm.at[pl.ds(base, per_w)], idx_v)
        pltpu.async_copy(table_hbm.at[idx_v], rows_v, sem).wait()
        pltpu.sync_copy(rows_v, out_hbm.at[pl.ds(base, per_w)])

    return gather_k


def _gru_body(sl_ref, emb_ref, wih_ref, whh_ref, bih_ref, bhh_ref,
              out_ref, last_ref, gi_ref, h_ref):
    t = pl.program_id(0)

    @pl.when(t == 0)
    def _():
        h_ref[...] = jnp.zeros_like(h_ref)

    gi_ref[...] = (
        jnp.dot(emb_ref[...], wih_ref[...], preferred_element_type=jnp.float32)
        + bih_ref[...]
    )

    whh = whh_ref[...]
    bhn = bhh_ref[...]
    sl = sl_ref[...]

    def one_step(j, h):
        gi = gi_ref[pl.ds(j * B, B), :]
        gh = jnp.dot(h.astype(jnp.bfloat16), whh,
                     preferred_element_type=jnp.float32)
        r = jax.nn.sigmoid(gi[:, 0:H] + gh[:, 0:H])
        z = jax.nn.sigmoid(gi[:, H:2 * H] + gh[:, H:2 * H])
        n = jnp.tanh(gi[:, 2 * H:3 * H] + r * (gh[:, 2 * H:3 * H] + bhn))
        h_new = n + z * (h - n)
        mt = ((t * TCH + j) < sl).astype(jnp.float32)
        out = mt * h_new
        out_ref[pl.ds(j, 1), :, :] = out[None]
        return h + mt * (h_new - h)

    def stepu(g, h):
        for u in range(UNROLL):
            h = one_step(g * UNROLL + u, h)
        return h

    def zero_step(g, _):
        out_ref[pl.ds(g, 1), :, :] = jnp.zeros((1, B, H), jnp.float32)
        return 0

    maxl = jnp.max(sl)
    live = jnp.clip(maxl - t * TCH, 0, TCH)
    nblk = (live + (UNROLL - 1)) // UNROLL
    h = lax.fori_loop(0, nblk, stepu, h_ref[...])
    lax.fori_loop(nblk * UNROLL, TCH, zero_step, 0)
    h_ref[...] = h
    last_ref[...] = h


def _gru_call(sl_b, emb_tm, wih_t, whh_t, bih, bhh, interpret=False):
    return pl.pallas_call(
        _gru_body,
        grid=(NT,),
        in_specs=[
            pl.BlockSpec((B, H), lambda t: (0, 0)),
            pl.BlockSpec((TCH * B, D), lambda t: (t, 0)),
            pl.BlockSpec((D, 3 * H), lambda t: (0, 0)),
            pl.BlockSpec((H, 3 * H), lambda t: (0, 0)),
            pl.BlockSpec((1, 3 * H), lambda t: (0, 0)),
            pl.BlockSpec((1, H), lambda t: (0, 0)),
        ],
        out_specs=(
            pl.BlockSpec((TCH, B, H), lambda t: (t, 0, 0)),
            pl.BlockSpec((B, H), lambda t: (0, 0)),
        ),
        out_shape=(
            jax.ShapeDtypeStruct((L, B, H), jnp.float32),
            jax.ShapeDtypeStruct((B, H), jnp.float32),
        ),
        scratch_shapes=[
            pltpu.VMEM((TCH * B, 3 * H), jnp.float32),
            pltpu.VMEM((B, H), jnp.float32),
        ],
        interpret=interpret,
    )(sl_b, emb_tm, wih_t, whh_t, bih, bhh)


def kernel(enc_inputs, sequence_length, current_input, embedding,
           W_ih, W_hh, b_ih, b_hh):
    del current_input
    idx_tm = jnp.swapaxes(enc_inputs, 0, 1).reshape(-1).astype(jnp.int32)
    emb_tm = _make_sc_gather(B * L, D)(embedding, idx_tm)
    sl_b = jnp.broadcast_to(
        sequence_length.astype(jnp.int32)[:, None], (B, H))
    bias_in = jnp.concatenate(
        [b_ih[0:2 * H] + b_hh[0:2 * H], b_ih[2 * H:3 * H]])[None, :]
    out_tm, last = _gru_call(sl_b, emb_tm, W_ih.T,
                             W_hh.T.astype(jnp.bfloat16),
                             bias_in, b_hh[None, 2 * H:3 * H])
    return jnp.swapaxes(out_tm, 0, 1), last


# speedup vs baseline: 1.0001x; 1.0001x over previous
import functools

import jax
import jax.numpy as jnp
from jax import lax
from jax.experimental import pallas as pl
from jax.experimental.pallas import tpu as pltpu
from jax.experimental.pallas import tpu_sc as plsc

B, L, V, D, H = 16, 512, 32000, 256, 256
TCH = 128
NT = L // TCH
UNROLL = 16


@functools.lru_cache(maxsize=None)
def _make_sc_gather(n_idx, d):
    info = plsc.get_sparse_core_info()
    nw = info.num_cores * info.num_subcores
    per_w = n_idx // nw
    mesh = plsc.VectorSubcoreMesh(core_axis_name="c", subcore_axis_name="s")

    @functools.partial(
        pl.kernel,
        mesh=mesh,
        out_type=jax.ShapeDtypeStruct((n_idx, d), jnp.float32),
        scratch_types=[
            pltpu.VMEM((per_w,), jnp.int32),
            pltpu.VMEM((per_w, d), jnp.float32),
            pltpu.SemaphoreType.DMA,
        ],
    )
    def gather_k(table_hbm, idx_hbm, out_hbm, idx_v, rows_v, sem):
        wid = lax.axis_index("s") * info.num_cores + lax.axis_index("c")
        base = wid * per_w
        pltpu.sync_copy(idx_hbm.at[pl.ds(base, per_w)], idx_v)
        pltpu.async_copy(table_hbm.at[idx_v], rows_v, sem).wait()
        pltpu.sync_copy(rows_v, out_hbm.at[pl.ds(base, per_w)])

    return gather_k


def _gru_body(sl_ref, emb_ref, wih_ref, whh_ref, bih_ref, bhh_ref,
              out_ref, last_ref, gi_ref, h_ref):
    t = pl.program_id(0)

    @pl.when(t == 0)
    def _():
        h_ref[...] = jnp.zeros_like(h_ref)

    gi_ref[...] = (
        jnp.dot(emb_ref[...], wih_ref[...], preferred_element_type=jnp.float32)
        + bih_ref[...]
    )

    whh = whh_ref[...]
    bhn = bhh_ref[...]
    sl = sl_ref[...]

    def one_step(j, h):
        gi = gi_ref[pl.ds(j * B, B), :]
        gh = jnp.dot(h.astype(jnp.bfloat16), whh,
                     preferred_element_type=jnp.float32)
        r = jax.nn.sigmoid(gi[:, 0:H] + gh[:, 0:H])
        z = jax.nn.sigmoid(gi[:, H:2 * H] + gh[:, H:2 * H])
        n = jnp.tanh(gi[:, 2 * H:3 * H] + r * (gh[:, 2 * H:3 * H] + bhn))
        h_new = n + z * (h - n)
        mt = ((t * TCH + j) < sl).astype(jnp.float32)
        out = mt * h_new
        out_ref[pl.ds(j, 1), :, :] = out[None]
        return h + mt * (h_new - h)

    def stepu(g, h):
        for u in range(UNROLL):
            h = one_step(g * UNROLL + u, h)
        return h

    def zero_step(g, _):
        out_ref[pl.ds(g, 1), :, :] = jnp.zeros((1, B, H), jnp.float32)
        return 0

    maxl = jnp.max(sl)
    live = jnp.clip(maxl - t * TCH, 0, TCH)
    nblk = (live + (UNROLL - 1)) // UNROLL
    h = lax.fori_loop(0, nblk, stepu, h_ref[...])
    lax.fori_loop(nblk * UNROLL, TCH, zero_step, 0)
    h_ref[...] = h
    last_ref[...] = h


def _gru_call(sl_b, emb_tm, wih_t, whh_t, bih, bhh, interpret=False):
    return pl.pallas_call(
        _gru_body,
        grid=(NT,),
        in_specs=[
            pl.BlockSpec((B, H), lambda t: (0, 0)),
            pl.BlockSpec((TCH * B, D), lambda t: (t, 0)),
            pl.BlockSpec((D, 3 * H), lambda t: (0, 0)),
            pl.BlockSpec((H, 3 * H), lambda t: (0, 0)),
            pl.BlockSpec((1, 3 * H), lambda t: (0, 0)),
            pl.BlockSpec((1, H), lambda t: (0, 0)),
        ],
        out_specs=(
            pl.BlockSpec((TCH, B, H), lambda t: (t, 0, 0)),
            pl.BlockSpec((B, H), lambda t: (0, 0)),
        ),
        out_shape=(
            jax.ShapeDtypeStruct((L, B, H), jnp.float32),
            jax.ShapeDtypeStruct((B, H), jnp.float32),
        ),
        scratch_shapes=[
            pltpu.VMEM((TCH * B, 3 * H), jnp.float32),
            pltpu.VMEM((B, H), jnp.float32),
        ],
        interpret=interpret,
    )(sl_b, emb_tm, wih_t, whh_t, bih, bhh)


def kernel(enc_inputs, sequence_length, current_input, embedding,
           W_ih, W_hh, b_ih, b_hh):
    del current_input
    idx_tm = jnp.swapaxes(enc_inputs, 0, 1).reshape(-1).astype(jnp.int32)
    emb_tm = _make_sc_gather(B * L, D)(embedding, idx_tm)
    sl_b = jnp.broadcast_to(
        sequence_length.astype(jnp.int32)[:, None], (B, H))
    bias_in = jnp.concatenate(
        [b_ih[0:2 * H] + b_hh[0:2 * H], b_ih[2 * H:3 * H]])[None, :]
    out_tm, last = _gru_call(sl_b, emb_tm, W_ih.T,
                             W_hh.T.astype(jnp.bfloat16),
                             bias_in, b_hh[None, 2 * H:3 * H])
    return jnp.swapaxes(out_tm, 0, 1), last


# speedup vs baseline: 1.0640x; 1.0639x over previous
import functools

import jax
import jax.numpy as jnp
from jax import lax
from jax.experimental import pallas as pl
from jax.experimental.pallas import tpu as pltpu
from jax.experimental.pallas import tpu_sc as plsc

B, L, V, D, H = 16, 512, 32000, 256, 256
TCH = 128
NT = L // TCH
UNROLL = 8


@functools.lru_cache(maxsize=None)
def _make_sc_gather(n_idx, d):
    info = plsc.get_sparse_core_info()
    nw = info.num_cores * info.num_subcores
    per_w = n_idx // nw
    mesh = plsc.VectorSubcoreMesh(core_axis_name="c", subcore_axis_name="s")

    @functools.partial(
        pl.kernel,
        mesh=mesh,
        out_type=jax.ShapeDtypeStruct((n_idx, d), jnp.float32),
        scratch_types=[
            pltpu.VMEM((per_w,), jnp.int32),
            pltpu.VMEM((per_w, d), jnp.float32),
            pltpu.SemaphoreType.DMA,
        ],
    )
    def gather_k(table_hbm, idx_hbm, out_hbm, idx_v, rows_v, sem):
        wid = lax.axis_index("s") * info.num_cores + lax.axis_index("c")
        base = wid * per_w
        pltpu.sync_copy(idx_hbm.at[pl.ds(base, per_w)], idx_v)
        pltpu.async_copy(table_hbm.at[idx_v], rows_v, sem).wait()
        pltpu.sync_copy(rows_v, out_hbm.at[pl.ds(base, per_w)])

    return gather_k


def _gru_body(sl_ref, emb_ref, wih_ref, whh_ref, bih_ref, bhh_ref,
              out_ref, last_ref, gi_ref, h_ref):
    t = pl.program_id(0)

    @pl.when(t == 0)
    def _():
        h_ref[...] = jnp.zeros_like(h_ref)

    gi_ref[...] = (
        jnp.dot(emb_ref[...], wih_ref[...], preferred_element_type=jnp.float32)
        + bih_ref[...]
    )

    whh = whh_ref[...]
    bhn = bhh_ref[...]
    sl = sl_ref[...]

    def one_step(j, h):
        gi = gi_ref[pl.ds(j * B, B), :]
        gh = jnp.dot(h.astype(jnp.bfloat16), whh,
                     preferred_element_type=jnp.float32)
        r = jax.nn.sigmoid(gi[:, 0:H] + gh[:, 0:H])
        z = jax.nn.sigmoid(gi[:, H:2 * H] + gh[:, H:2 * H])
        n = jnp.tanh(gi[:, 2 * H:3 * H] + r * (gh[:, 2 * H:3 * H] + bhn))
        h_new = n + z * (h - n)
        mt = ((t * TCH + j) < sl).astype(jnp.float32)
        out = mt * h_new
        out_ref[:, pl.ds(j, 1), :] = out[:, None, :]
        return h + mt * (h_new - h)

    def stepu(g, h):
        for u in range(UNROLL):
            h = one_step(g * UNROLL + u, h)
        return h

    def zero_step(g, _):
        out_ref[:, pl.ds(g, 1), :] = jnp.zeros((B, 1, H), jnp.float32)
        return 0

    maxl = jnp.max(sl)
    live = jnp.clip(maxl - t * TCH, 0, TCH)
    nblk = (live + (UNROLL - 1)) // UNROLL
    h = lax.fori_loop(0, nblk, stepu, h_ref[...])
    lax.fori_loop(nblk * UNROLL, TCH, zero_step, 0)
    h_ref[...] = h
    last_ref[...] = h


def _gru_call(sl_b, emb_tm, wih_t, whh_t, bih, bhh, interpret=False):
    return pl.pallas_call(
        _gru_body,
        grid=(NT,),
        in_specs=[
            pl.BlockSpec((B, H), lambda t: (0, 0)),
            pl.BlockSpec((TCH * B, D), lambda t: (t, 0)),
            pl.BlockSpec((D, 3 * H), lambda t: (0, 0)),
            pl.BlockSpec((H, 3 * H), lambda t: (0, 0)),
            pl.BlockSpec((1, 3 * H), lambda t: (0, 0)),
            pl.BlockSpec((1, H), lambda t: (0, 0)),
        ],
        out_specs=(
            pl.BlockSpec((B, TCH, H), lambda t: (0, t, 0)),
            pl.BlockSpec((B, H), lambda t: (0, 0)),
        ),
        out_shape=(
            jax.ShapeDtypeStruct((B, L, H), jnp.float32),
            jax.ShapeDtypeStruct((B, H), jnp.float32),
        ),
        scratch_shapes=[
            pltpu.VMEM((TCH * B, 3 * H), jnp.float32),
            pltpu.VMEM((B, H), jnp.float32),
        ],
        interpret=interpret,
    )(sl_b, emb_tm, wih_t, whh_t, bih, bhh)


def kernel(enc_inputs, sequence_length, current_input, embedding,
           W_ih, W_hh, b_ih, b_hh):
    del current_input
    idx_tm = jnp.swapaxes(enc_inputs, 0, 1).reshape(-1).astype(jnp.int32)
    emb_tm = _make_sc_gather(B * L, D)(embedding, idx_tm)
    sl_b = jnp.broadcast_to(
        sequence_length.astype(jnp.int32)[:, None], (B, H))
    bias_in = jnp.concatenate(
        [b_ih[0:2 * H] + b_hh[0:2 * H], b_ih[2 * H:3 * H]])[None, :]
    out, last = _gru_call(sl_b, emb_tm, W_ih.T,
                             W_hh.T.astype(jnp.bfloat16),
                             bias_in, b_hh[None, 2 * H:3 * H])
    return out, last
